# Initial kernel scaffold; baseline (speedup 1.0000x reference)
import jax
import jax.numpy as jnp
from jax.experimental import pallas as pl


def kernel(x, edge_index, W1, b1, W2, b2):
    raise NotImplementedError("write your pallas kernel here")


# speedup vs baseline: 36.8026x; 36.8026x over previous
import functools

import jax
import jax.numpy as jnp
from jax import lax
from jax.experimental import pallas as pl
from jax.experimental.pallas import tpu as pltpu
from jax.experimental.pallas import tpu_sc as plsc

_N = 10000
_E = 320000
_F_IN = 128
_HID = 16

_NC = 2
_NS = 16
_NW = _NC * _NS
_EPW = _E // _NW
_NCH = 80
_CH = 125
_NPAD = 10112
_STRIPE = _NPAD // _NS

_BN = 1000


def _sc_mesh():
    return plsc.VectorSubcoreMesh(
        core_axis_name="c", subcore_axis_name="s",
        num_cores=_NC, num_subcores=_NS)


def _sc_degree(dst_flat):
    @functools.partial(
        pl.kernel,
        out_type=jax.ShapeDtypeStruct((_NW, _NPAD), jnp.float32),
        mesh=_sc_mesh(),
        scratch_types=[
            pltpu.VMEM((_EPW,), jnp.int32),
            pltpu.VMEM((_NPAD,), jnp.float32),
        ],
        compiler_params=pltpu.CompilerParams(needs_layout_passes=False),
    )
    def deg_kernel(dst_hbm, out_hbm, dst_v, deg_v):
        cid = lax.axis_index("c")
        sid = lax.axis_index("s")
        wid = sid * _NC + cid
        pltpu.sync_copy(dst_hbm.at[wid], dst_v)

        zero16 = jnp.zeros((16,), jnp.float32)

        @pl.loop(0, _NPAD // 16)
        def _zero(i):
            deg_v[pl.ds(i * 16, 16)] = zero16

        one16 = jnp.ones((16,), jnp.float32)

        @pl.loop(0, _EPW // 16)
        def _hist(i):
            idx = dst_v[pl.ds(i * 16, 16)]
            plsc.addupdate_scatter(deg_v, [idx], one16)

        pltpu.sync_copy(deg_v, out_hbm.at[wid])

    return deg_kernel(dst_flat)


def _sc_aggregate(y, src3, dst3, zeros_pad):
    @functools.partial(
        pl.kernel,
        out_type=jax.ShapeDtypeStruct((_NC, _NPAD, _HID), jnp.float32),
        mesh=_sc_mesh(),
        scratch_types=[
            pltpu.VMEM((_NCH, _CH), jnp.int32),
            pltpu.VMEM((_NCH, _CH), jnp.int32),
            pltpu.VMEM((_CH, _HID), jnp.float32),
            pltpu.VMEM((_STRIPE, _HID), jnp.float32),
            pltpu.VMEM_SHARED((_NPAD, _HID), jnp.float32),
            pltpu.SemaphoreType.DMA,
        ],
        compiler_params=pltpu.CompilerParams(use_tc_tiling_on_sc=False),
    )
    def agg_kernel(y_hbm, src_hbm, dst_hbm, zero_hbm, out_hbm,
                   src_v, dst_v, rows_v, stage_v, acc, sem):
        cid = lax.axis_index("c")
        sid = lax.axis_index("s")
        wid = sid * _NC + cid

        pltpu.sync_copy(src_hbm.at[wid], src_v)
        pltpu.sync_copy(dst_hbm.at[wid], dst_v)
        pltpu.sync_copy(zero_hbm.at[pl.ds(sid * _STRIPE, _STRIPE)],
                        acc.at[pl.ds(sid * _STRIPE, _STRIPE)])
        plsc.subcore_barrier()

        @pl.loop(0, _NCH)
        def _chunk(j):
            pltpu.async_copy(y_hbm.at[src_v.at[j]], rows_v, sem).wait()
            pltpu.sync_copy(rows_v, acc.at[dst_v.at[j]], add=True)

        plsc.subcore_barrier()
        pltpu.sync_copy(acc.at[pl.ds(sid * _STRIPE, _STRIPE)], stage_v)
        pltpu.sync_copy(stage_v,
                        out_hbm.at[cid, pl.ds(sid * _STRIPE, _STRIPE)])

    return agg_kernel(y, src3, dst3, zeros_pad)


def _tc_layer1(x, W1, deg_part):
    def body(x_ref, w_ref, deg_ref, y_ref, dinv_ref):
        deg = jnp.sum(deg_ref[:, :_N], axis=0) + 1.0
        dinv = lax.rsqrt(deg)
        xw = jnp.dot(x_ref[...], w_ref[...], preferred_element_type=jnp.float32)
        y_ref[...] = xw * dinv[:, None]
        dinv_ref[...] = jnp.broadcast_to(dinv[:, None], (_N, _HID))

    return pl.pallas_call(
        body,
        out_shape=[
            jax.ShapeDtypeStruct((_N, _HID), jnp.float32),
            jax.ShapeDtypeStruct((_N, _HID), jnp.float32),
        ],
    )(x, W1, deg_part)


def _tc_layer2(p, y1, dinvb, b1r, W2):
    def body(p_ref, y_ref, d_ref, b_ref, w_ref, out_ref):
        d = d_ref[...]
        h = jnp.maximum(d * (p_ref[0] + p_ref[1] + y_ref[...]) + b_ref[...], 0.0)
        out_ref[...] = jnp.dot(
            h, w_ref[...], preferred_element_type=jnp.float32) * d

    return pl.pallas_call(
        body,
        out_shape=jax.ShapeDtypeStruct((_N, _HID), jnp.float32),
    )(p, y1, dinvb, b1r, W2)


def _tc_layer3(p, y2, dinvb, b2r):
    def body(p_ref, y_ref, d_ref, b_ref, out_ref):
        o = d_ref[...] * (p_ref[0] + p_ref[1] + y_ref[...]) + b_ref[...]
        m = jnp.max(o, axis=1, keepdims=True)
        s = o - m
        out_ref[...] = s - jnp.log(jnp.sum(jnp.exp(s), axis=1, keepdims=True))

    return pl.pallas_call(
        body,
        out_shape=jax.ShapeDtypeStruct((_N, _HID), jnp.float32),
    )(p, y2, dinvb, b2r)


def kernel(x, edge_index, W1, b1, W2, b2):
    src3 = edge_index[0].reshape(_NW, _NCH, _CH)
    dst3 = edge_index[1].reshape(_NW, _NCH, _CH)
    dst_flat = edge_index[1].reshape(_NW, _EPW)
    zeros_pad = jnp.zeros((_NPAD, _HID), jnp.float32)

    deg_part = _sc_degree(dst_flat)
    y1, dinvb = _tc_layer1(x, W1, deg_part)
    p1 = _sc_aggregate(y1, src3, dst3, zeros_pad)[:, :_N]
    y2 = _tc_layer2(p1, y1, dinvb, b1.reshape(1, _HID), W2)
    p2 = _sc_aggregate(y2, src3, dst3, zeros_pad)[:, :_N]
    return _tc_layer3(p2, y2, dinvb, b2.reshape(1, _HID))


# speedup vs baseline: 59.9863x; 1.6299x over previous
import functools

import jax
import jax.numpy as jnp
from jax import lax
from jax.experimental import pallas as pl
from jax.experimental.pallas import tpu as pltpu
from jax.experimental.pallas import tpu_sc as plsc

_N = 10000
_E = 320000
_F_IN = 128
_HID = 16

_NC = 2
_NS = 16
_NW = _NC * _NS
_EPW = _E // _NW
_NCH = 80
_CH = 125
_NPAD = 10112
_STRIPE = _NPAD // _NS

_BN = 1000


def _sc_mesh():
    return plsc.VectorSubcoreMesh(
        core_axis_name="c", subcore_axis_name="s",
        num_cores=_NC, num_subcores=_NS)


def _sc_degree(dst_flat):
    @functools.partial(
        pl.kernel,
        out_type=jax.ShapeDtypeStruct((_NW, _NPAD), jnp.float32),
        mesh=_sc_mesh(),
        scratch_types=[
            pltpu.VMEM((_EPW,), jnp.int32),
            pltpu.VMEM((_NPAD,), jnp.float32),
        ],
        compiler_params=pltpu.CompilerParams(needs_layout_passes=False),
    )
    def deg_kernel(dst_hbm, out_hbm, dst_v, deg_v):
        cid = lax.axis_index("c")
        sid = lax.axis_index("s")
        wid = sid * _NC + cid
        pltpu.sync_copy(dst_hbm.at[wid], dst_v)

        zero16 = jnp.zeros((16,), jnp.float32)

        @pl.loop(0, _NPAD // 16)
        def _zero(i):
            deg_v[pl.ds(i * 16, 16)] = zero16

        one16 = jnp.ones((16,), jnp.float32)

        @pl.loop(0, _EPW // 16)
        def _hist(i):
            idx = dst_v[pl.ds(i * 16, 16)]
            plsc.addupdate_scatter(deg_v, [idx], one16)

        pltpu.sync_copy(deg_v, out_hbm.at[wid])

    return deg_kernel(dst_flat)


def _sc_aggregate(y, src3, dst3, zeros_pad):
    nbuf = 8
    pre = 4

    @functools.partial(
        pl.kernel,
        out_type=jax.ShapeDtypeStruct((_NC, _NPAD, _HID), jnp.float32),
        mesh=_sc_mesh(),
        scratch_types=[
            pltpu.VMEM((_NCH, _CH), jnp.int32),
            pltpu.VMEM((_NCH, _CH), jnp.int32),
            [pltpu.VMEM((_CH, _HID), jnp.float32) for _ in range(nbuf)],
            pltpu.VMEM((_STRIPE, _HID), jnp.float32),
            pltpu.VMEM_SHARED((_NPAD, _HID), jnp.float32),
            [pltpu.SemaphoreType.DMA for _ in range(nbuf)],
            [pltpu.SemaphoreType.DMA for _ in range(nbuf)],
        ],
        compiler_params=pltpu.CompilerParams(use_tc_tiling_on_sc=False),
    )
    def agg_kernel(y_hbm, src_hbm, dst_hbm, zero_hbm, out_hbm,
                   src_v, dst_v, rows_v, stage_v, acc, gsem, ssem):
        cid = lax.axis_index("c")
        sid = lax.axis_index("s")
        wid = sid * _NC + cid

        pltpu.sync_copy(src_hbm.at[wid], src_v)
        pltpu.sync_copy(dst_hbm.at[wid], dst_v)
        pltpu.sync_copy(zero_hbm.at[pl.ds(sid * _STRIPE, _STRIPE)],
                        acc.at[pl.ds(sid * _STRIPE, _STRIPE)])
        plsc.subcore_barrier()

        for b in range(pre):
            pltpu.async_copy(y_hbm.at[src_v.at[b]], rows_v[b], gsem[b])

        @pl.loop(0, _NCH // nbuf)
        def _blk(jj):
            for b in range(nbuf):
                c = jj * nbuf + b
                pltpu.make_async_copy(y_hbm.at[src_v.at[c]],
                                      rows_v[b], gsem[b]).wait()
                pltpu.async_copy(rows_v[b], acc.at[dst_v.at[c]],
                                 ssem[b], add=True)
                b2 = (b + pre) % nbuf

                @pl.when(c >= pre)
                def _drain():
                    pltpu.make_async_copy(rows_v[b2], acc.at[dst_v.at[c]],
                                          ssem[b2]).wait()

                @pl.when(c < _NCH - pre)
                def _prefetch():
                    pltpu.async_copy(y_hbm.at[src_v.at[c + pre]],
                                     rows_v[b2], gsem[b2])

        for b in range(pre):
            b2 = (b + pre) % nbuf
            pltpu.make_async_copy(rows_v[b2], acc.at[dst_v.at[0]],
                                  ssem[b2]).wait()

        plsc.subcore_barrier()
        pltpu.sync_copy(acc.at[pl.ds(sid * _STRIPE, _STRIPE)], stage_v)
        pltpu.sync_copy(stage_v,
                        out_hbm.at[cid, pl.ds(sid * _STRIPE, _STRIPE)])

    return agg_kernel(y, src3, dst3, zeros_pad)


def _tc_layer1(x, W1, deg_part):
    def body(x_ref, w_ref, deg_ref, y_ref, dinv_ref):
        deg = jnp.sum(deg_ref[:, :_N], axis=0) + 1.0
        dinv = lax.rsqrt(deg)
        xw = jnp.dot(x_ref[...], w_ref[...], preferred_element_type=jnp.float32)
        y_ref[...] = xw * dinv[:, None]
        dinv_ref[...] = jnp.broadcast_to(dinv[:, None], (_N, _HID))

    return pl.pallas_call(
        body,
        out_shape=[
            jax.ShapeDtypeStruct((_N, _HID), jnp.float32),
            jax.ShapeDtypeStruct((_N, _HID), jnp.float32),
        ],
    )(x, W1, deg_part)


def _tc_layer2(p, y1, dinvb, b1r, W2):
    def body(p_ref, y_ref, d_ref, b_ref, w_ref, out_ref):
        d = d_ref[...]
        h = jnp.maximum(d * (p_ref[0] + p_ref[1] + y_ref[...]) + b_ref[...], 0.0)
        out_ref[...] = jnp.dot(
            h, w_ref[...], preferred_element_type=jnp.float32) * d

    return pl.pallas_call(
        body,
        out_shape=jax.ShapeDtypeStruct((_N, _HID), jnp.float32),
    )(p, y1, dinvb, b1r, W2)


def _tc_layer3(p, y2, dinvb, b2r):
    def body(p_ref, y_ref, d_ref, b_ref, out_ref):
        o = d_ref[...] * (p_ref[0] + p_ref[1] + y_ref[...]) + b_ref[...]
        m = jnp.max(o, axis=1, keepdims=True)
        s = o - m
        out_ref[...] = s - jnp.log(jnp.sum(jnp.exp(s), axis=1, keepdims=True))

    return pl.pallas_call(
        body,
        out_shape=jax.ShapeDtypeStruct((_N, _HID), jnp.float32),
    )(p, y2, dinvb, b2r)


def kernel(x, edge_index, W1, b1, W2, b2):
    src3 = edge_index[0].reshape(_NW, _NCH, _CH)
    dst3 = edge_index[1].reshape(_NW, _NCH, _CH)
    dst_flat = edge_index[1].reshape(_NW, _EPW)
    zeros_pad = jnp.zeros((_NPAD, _HID), jnp.float32)

    deg_part = _sc_degree(dst_flat)
    y1, dinvb = _tc_layer1(x, W1, deg_part)
    p1 = _sc_aggregate(y1, src3, dst3, zeros_pad)[:, :_N]
    y2 = _tc_layer2(p1, y1, dinvb, b1.reshape(1, _HID), W2)
    p2 = _sc_aggregate(y2, src3, dst3, zeros_pad)[:, :_N]
    return _tc_layer3(p2, y2, dinvb, b2.reshape(1, _HID))


# speedup vs baseline: 84.6927x; 1.4119x over previous
import functools

import jax
import jax.numpy as jnp
from jax import lax
from jax.experimental import pallas as pl
from jax.experimental.pallas import tpu as pltpu
from jax.experimental.pallas import tpu_sc as plsc

_N = 10000
_E = 320000
_F_IN = 128
_HID = 16

_NC = 2
_NS = 16
_NW = _NC * _NS
_EPW = _E // _NW
_NCH = 80
_CH = 125
_NPAD = 10112
_STRIPE = _NPAD // _NS
_PK = _NPAD // 8
_PKN = _N // 8


def _sc_mesh():
    return plsc.VectorSubcoreMesh(
        core_axis_name="c", subcore_axis_name="s",
        num_cores=_NC, num_subcores=_NS)


def _sc_degree(edge_index):
    @functools.partial(
        pl.kernel,
        out_type=[
            jax.ShapeDtypeStruct((_NW, _NPAD), jnp.float32),
            jax.ShapeDtypeStruct((_NW, _EPW), jnp.int32),
            jax.ShapeDtypeStruct((_NW, _EPW), jnp.int32),
        ],
        mesh=_sc_mesh(),
        scratch_types=[
            pltpu.VMEM((_EPW,), jnp.int32),
            pltpu.VMEM((_EPW,), jnp.int32),
            pltpu.VMEM((_NPAD,), jnp.float32),
        ],
        compiler_params=pltpu.CompilerParams(
            needs_layout_passes=False, use_tc_tiling_on_sc=False),
    )
    def deg_kernel(e_hbm, deg_hbm, src_hbm, dst_hbm, src_v, dst_v, deg_v):
        cid = lax.axis_index("c")
        sid = lax.axis_index("s")
        wid = sid * _NC + cid
        base = wid * _EPW
        pltpu.sync_copy(e_hbm.at[0, pl.ds(base, _EPW)], src_v)
        pltpu.sync_copy(e_hbm.at[1, pl.ds(base, _EPW)], dst_v)

        zero16 = jnp.zeros((16,), jnp.float32)

        @pl.loop(0, _NPAD // 16)
        def _zero(i):
            deg_v[pl.ds(i * 16, 16)] = zero16

        one16 = jnp.ones((16,), jnp.float32)

        @pl.loop(0, _EPW // 16)
        def _hist(i):
            idx = dst_v[pl.ds(i * 16, 16)]
            plsc.addupdate_scatter(deg_v, [idx], one16)

        pltpu.sync_copy(deg_v, deg_hbm.at[wid])
        pltpu.sync_copy(src_v, src_hbm.at[wid])
        pltpu.sync_copy(dst_v, dst_hbm.at[wid])

    return deg_kernel(edge_index)


def _sc_aggregate(y, src3, dst3, zeros_pad):
    nbuf = 8
    pre = 4

    @functools.partial(
        pl.kernel,
        out_type=jax.ShapeDtypeStruct((_NC, _NPAD, _HID), jnp.float32),
        mesh=_sc_mesh(),
        scratch_types=[
            pltpu.VMEM((_NCH, _CH), jnp.int32),
            pltpu.VMEM((_NCH, _CH), jnp.int32),
            [pltpu.VMEM((_CH, _HID), jnp.float32) for _ in range(nbuf)],
            pltpu.VMEM((_STRIPE, _HID), jnp.float32),
            pltpu.VMEM_SHARED((_NPAD, _HID), jnp.float32),
            [pltpu.SemaphoreType.DMA for _ in range(nbuf)],
            [pltpu.SemaphoreType.DMA for _ in range(nbuf)],
        ],
        compiler_params=pltpu.CompilerParams(use_tc_tiling_on_sc=False),
    )
    def agg_kernel(y_hbm, src_hbm, dst_hbm, zero_hbm, out_hbm,
                   src_v, dst_v, rows_v, stage_v, acc, gsem, ssem):
        cid = lax.axis_index("c")
        sid = lax.axis_index("s")
        wid = sid * _NC + cid

        pltpu.sync_copy(src_hbm.at[wid], src_v)
        pltpu.sync_copy(dst_hbm.at[wid], dst_v)
        pltpu.sync_copy(zero_hbm.at[pl.ds(sid * _STRIPE, _STRIPE)],
                        acc.at[pl.ds(sid * _STRIPE, _STRIPE)])
        plsc.subcore_barrier()

        for b in range(pre):
            pltpu.async_copy(y_hbm.at[src_v.at[b]], rows_v[b], gsem[b])

        @pl.loop(0, _NCH // nbuf)
        def _blk(jj):
            for b in range(nbuf):
                c = jj * nbuf + b
                pltpu.make_async_copy(y_hbm.at[src_v.at[c]],
                                      rows_v[b], gsem[b]).wait()
                pltpu.async_copy(rows_v[b], acc.at[dst_v.at[c]],
                                 ssem[b], add=True)
                b2 = (b + pre) % nbuf

                @pl.when(c >= pre)
                def _drain():
                    pltpu.make_async_copy(rows_v[b2], acc.at[dst_v.at[c]],
                                          ssem[b2]).wait()

                @pl.when(c < _NCH - pre)
                def _prefetch():
                    pltpu.async_copy(y_hbm.at[src_v.at[c + pre]],
                                     rows_v[b2], gsem[b2])

        for b in range(pre):
            b2 = (b + pre) % nbuf
            pltpu.make_async_copy(rows_v[b2], acc.at[dst_v.at[0]],
                                  ssem[b2]).wait()

        plsc.subcore_barrier()
        pltpu.sync_copy(acc.at[pl.ds(sid * _STRIPE, _STRIPE)], stage_v)
        pltpu.sync_copy(stage_v,
                        out_hbm.at[cid, pl.ds(sid * _STRIPE, _STRIPE)])

    return agg_kernel(y, src3, dst3, zeros_pad)


def _tc_layer1(xp, Wb1, deg_part):
    def body(x_ref, w_ref, deg_ref, xw_ref, dinv_ref):
        deg = jnp.sum(deg_ref[...], axis=0, keepdims=True)
        dinv_ref[...] = lax.rsqrt(deg + 1.0)
        xw = jnp.dot(x_ref[...], w_ref[...], preferred_element_type=jnp.float32)
        xw_ref[0:_PKN] = xw
        xw_ref[_PKN:_PK] = jnp.zeros((_PK - _PKN, 128), jnp.float32)

    return pl.pallas_call(
        body,
        out_shape=[
            jax.ShapeDtypeStruct((_PK, 128), jnp.float32),
            jax.ShapeDtypeStruct((1, _NPAD), jnp.float32),
        ],
    )(xp, Wb1, deg_part)


def _tc_layer2(p, y1p, dinvp, b1p, Wb2):
    def body(p_ref, y_ref, d_ref, b_ref, w_ref, out_ref):
        d = d_ref[...]
        h = jnp.maximum(d * (p_ref[0] + p_ref[1] + y_ref[...]) + b_ref[...],
                        0.0)
        out_ref[...] = jnp.dot(
            h, w_ref[...], preferred_element_type=jnp.float32) * d

    return pl.pallas_call(
        body,
        out_shape=jax.ShapeDtypeStruct((_PK, 128), jnp.float32),
    )(p, y1p, dinvp, b1p, Wb2)


def _tc_layer3(p, y2p, dinvp, b2p, R):
    def body(p_ref, y_ref, d_ref, b_ref, r_ref, out_ref):
        o = d_ref[...] * (p_ref[0] + p_ref[1] + y_ref[...]) + b_ref[...]
        s = o - jnp.max(o)
        e = jnp.exp(s)
        r = r_ref[...]
        t = lax.dot_general(e, r, (((1,), (1,)), ((), ())),
                            preferred_element_type=jnp.float32)
        out_ref[...] = s - jnp.dot(jnp.log(t), r,
                                   preferred_element_type=jnp.float32)

    return pl.pallas_call(
        body,
        out_shape=jax.ShapeDtypeStruct((_PK, 128), jnp.float32),
    )(p, y2p, dinvp, b2p, R)


def kernel(x, edge_index, W1, b1, W2, b2):
    eye8 = jnp.eye(8, dtype=jnp.float32)
    Wb1 = jnp.kron(eye8, W1)
    Wb2 = jnp.kron(eye8, W2)
    R = jnp.kron(eye8, jnp.ones((1, 16), jnp.float32))
    b1p = jnp.tile(b1, 8).reshape(1, 128)
    b2p = jnp.tile(b2, 8).reshape(1, 128)
    xp = x.reshape(_PKN, 8 * _F_IN)
    zeros_pad = jnp.zeros((_NPAD, _HID), jnp.float32)

    deg_part, src_f, dst_f = _sc_degree(edge_index)
    src3 = src_f.reshape(_NW, _NCH, _CH)
    dst3 = dst_f.reshape(_NW, _NCH, _CH)

    xwp, dinv = _tc_layer1(xp, Wb1, deg_part)
    dinvp = jnp.broadcast_to(
        dinv.reshape(_PK, 8, 1), (_PK, 8, _HID)).reshape(_PK, 128)
    y1p = xwp * dinvp
    p1 = _sc_aggregate(y1p.reshape(_NPAD, _HID), src3, dst3, zeros_pad)
    y2p = _tc_layer2(p1.reshape(_NC, _PK, 128), y1p, dinvp, b1p, Wb2)
    p2 = _sc_aggregate(y2p.reshape(_NPAD, _HID), src3, dst3, zeros_pad)
    lsp = _tc_layer3(p2.reshape(_NC, _PK, 128), y2p, dinvp, b2p, R)
    return lsp[:_PKN].reshape(_N, _HID)
